# speedup vs baseline: 1.9663x; 1.9663x over previous
import jax
import jax.numpy as jnp
from jax.experimental import pallas as pl

BM = 512


def _body(x_ref, c_ref, t_ref, b_ref, o_ref):
    xb = x_ref[...]
    c = c_ref[...]
    x2 = jnp.sum(xb * xb, axis=1, keepdims=True)
    c2 = jnp.sum(c * c, axis=1)
    dot = jax.lax.dot_general(
        xb, c, (((1,), (1,)), ((), ())), preferred_element_type=jnp.float32
    )
    d2 = x2 + c2[None, :] - 2.0 * dot
    nn = jnp.argmin(d2, axis=1)
    k = d2.shape[1]
    oh = (jax.lax.broadcasted_iota(jnp.int32, (xb.shape[0], k), 1)
          == nn[:, None]).astype(jnp.float32)
    out = jax.lax.dot_general(
        oh, t_ref[...], (((1,), (0,)), ((), ())),
        preferred_element_type=jnp.float32,
    )
    o_ref[...] = out + b_ref[...]


def kernel(x, input_centroids, lookup_table_fc1, lookup_table_fc2,
           fc1_bias, fc2_bias):
    del lookup_table_fc1, fc1_bias
    b, s, d = x.shape
    n = b * s
    k = input_centroids.shape[0]
    o = lookup_table_fc2.shape[1]
    x_flat = x.reshape(n, d)
    out = pl.pallas_call(
        _body,
        grid=(n // BM,),
        in_specs=[
            pl.BlockSpec((BM, d), lambda i: (i, 0)),
            pl.BlockSpec((k, d), lambda i: (0, 0)),
            pl.BlockSpec((k, o), lambda i: (0, 0)),
            pl.BlockSpec((1, o), lambda i: (0, 0)),
        ],
        out_specs=pl.BlockSpec((BM, o), lambda i: (i, 0)),
        out_shape=jax.ShapeDtypeStruct((n, o), jnp.float32),
    )(x_flat, input_centroids, lookup_table_fc2, fc2_bias.reshape(1, o))
    return out.reshape(b, s, o)
